# Initial kernel scaffold; baseline (speedup 1.0000x reference)
import functools

import jax
import jax.numpy as jnp
from jax import lax
from jax.experimental import pallas as pl
from jax.experimental.pallas import tpu as pltpu
from jax.experimental.pallas import tpu_sc as plsc

B = 16384
F = 26
D = 32
TOTAL = B * F
NW = 32
PER_W = TOTAL // NW
CHUNK = 128
NCHUNK = PER_W // CHUNK

_mesh = plsc.VectorSubcoreMesh(core_axis_name="c", subcore_axis_name="s")


@functools.partial(
    pl.kernel,
    mesh=_mesh,
    out_type=jax.ShapeDtypeStruct((TOTAL, D), jnp.float32),
    scratch_types=[
        pltpu.VMEM((NCHUNK, CHUNK), jnp.int32),
        pltpu.VMEM((CHUNK, D), jnp.float32),
        pltpu.SemaphoreType.DMA,
    ],
)
def _lookup(idx_hbm, table_hbm, out_hbm, idx_v, rows_v, gsem):
    wid = lax.axis_index("s") * 2 + lax.axis_index("c")
    base = wid * PER_W
    pltpu.sync_copy(idx_hbm.at[pl.ds(wid * NCHUNK, NCHUNK)], idx_v)

    def body(c, _):
        pltpu.async_copy(table_hbm.at[idx_v.at[c]], rows_v, gsem).wait()
        pltpu.sync_copy(rows_v, out_hbm.at[pl.ds(base + c * CHUNK, CHUNK)])
        return 0

    lax.fori_loop(0, NCHUNK, body, 0)


def kernel(x, W):
    idx2d = x.reshape(TOTAL // CHUNK, CHUNK)
    out = _lookup(idx2d, W)
    return out.reshape(B, F, D)


# speedup vs baseline: 1.4367x; 1.4367x over previous
import functools

import jax
import jax.numpy as jnp
from jax import lax
from jax.experimental import pallas as pl
from jax.experimental.pallas import tpu as pltpu
from jax.experimental.pallas import tpu_sc as plsc

B = 16384
F = 26
D = 32
TOTAL = B * F
NW = 32
PER_W = TOTAL // NW
CHUNK = 128
NCHUNK = PER_W // CHUNK

_mesh = plsc.VectorSubcoreMesh(core_axis_name="c", subcore_axis_name="s")


@functools.partial(
    pl.kernel,
    mesh=_mesh,
    out_type=jax.ShapeDtypeStruct((TOTAL, D), jnp.float32),
    compiler_params=pltpu.CompilerParams(use_tc_tiling_on_sc=False),
    scratch_types=[
        pltpu.VMEM((NCHUNK, CHUNK), jnp.int32),
        pltpu.VMEM((CHUNK, D), jnp.float32),
        pltpu.SemaphoreType.DMA,
    ],
)
def _lookup(idx_hbm, table_hbm, out_hbm, idx_v, rows_v, gsem):
    wid = lax.axis_index("s") * 2 + lax.axis_index("c")
    base = wid * PER_W
    pltpu.sync_copy(idx_hbm.at[pl.ds(wid * NCHUNK, NCHUNK)], idx_v)

    def body(c, _):
        pltpu.async_copy(table_hbm.at[idx_v.at[c]], rows_v, gsem).wait()
        pltpu.sync_copy(rows_v, out_hbm.at[pl.ds(base + c * CHUNK, CHUNK)])
        return 0

    lax.fori_loop(0, NCHUNK, body, 0)


def kernel(x, W):
    idx2d = x.reshape(TOTAL // CHUNK, CHUNK)
    out = _lookup(idx2d, W)
    return out.reshape(B, F, D)


# speedup vs baseline: 1.5630x; 1.0879x over previous
import functools

import jax
import jax.numpy as jnp
from jax import lax
from jax.experimental import pallas as pl
from jax.experimental.pallas import tpu as pltpu
from jax.experimental.pallas import tpu_sc as plsc

B = 16384
F = 26
D = 32
TOTAL = B * F
NW = 32
PER_W = TOTAL // NW
CHUNK = 128
NCHUNK = PER_W // CHUNK
NBUF = 4
ROUNDS = NCHUNK // NBUF
assert NCHUNK % NBUF == 0 and ROUNDS % 2 == 0

_mesh = plsc.VectorSubcoreMesh(core_axis_name="c", subcore_axis_name="s")


@functools.partial(
    pl.kernel,
    mesh=_mesh,
    out_type=jax.ShapeDtypeStruct((TOTAL, D), jnp.float32),
    compiler_params=pltpu.CompilerParams(use_tc_tiling_on_sc=False),
    scratch_types=[
        pltpu.VMEM((NCHUNK, CHUNK), jnp.int32),
        pltpu.VMEM((2, NBUF, CHUNK, D), jnp.float32),
        pltpu.SemaphoreType.DMA((2, NBUF)),
        pltpu.SemaphoreType.DMA((2, NBUF)),
    ],
)
def _lookup(idx_hbm, table_hbm, out_hbm, idx_v, rows_v, gsem, osem):
    wid = lax.axis_index("s") * 2 + lax.axis_index("c")
    base = wid * PER_W
    pltpu.sync_copy(idx_hbm.at[pl.ds(wid * NCHUNK, NCHUNK)], idx_v)

    def gather_start(c, p, b):
        pltpu.async_copy(table_hbm.at[idx_v.at[c]], rows_v.at[p, b],
                         gsem.at[p, b])

    def gather_wait(p, b):
        pltpu.make_async_copy(table_hbm.at[idx_v.at[0]], rows_v.at[p, b],
                              gsem.at[p, b]).wait()

    def out_start(c, p, b):
        pltpu.async_copy(rows_v.at[p, b],
                         out_hbm.at[pl.ds(base + c * CHUNK, CHUNK)],
                         osem.at[p, b])

    def out_wait(p, b):
        pltpu.make_async_copy(rows_v.at[p, b],
                              out_hbm.at[pl.ds(base, CHUNK)],
                              osem.at[p, b]).wait()

    for b in range(NBUF):
        gather_start(b, 0, b)

    def body(t, _):
        for p in (0, 1):
            g = 2 * t + p
            q = 1 - p
            for b in range(NBUF):
                c = g * NBUF + b
                gather_wait(p, b)
                out_start(c, p, b)
            for b in range(NBUF):
                if p == 0:
                    @pl.when(t > 0)
                    def _():
                        out_wait(q, b)
                    gather_start((g + 1) * NBUF + b, q, b)
                else:
                    @pl.when(2 * t + 2 < ROUNDS)
                    def _():
                        out_wait(q, b)
                        gather_start((g + 1) * NBUF + b, q, b)
        return 0

    lax.fori_loop(0, ROUNDS // 2, body, 0)
    for p in (0, 1):
        for b in range(NBUF):
            out_wait(p, b)


def kernel(x, W):
    idx2d = x.reshape(TOTAL // CHUNK, CHUNK)
    out = _lookup(idx2d, W)
    return out.reshape(B, F, D)
